# Initial kernel scaffold; baseline (speedup 1.0000x reference)
import jax
import jax.numpy as jnp
from jax.experimental import pallas as pl


def kernel(loc_data, conf_data, priors, targets):
    raise NotImplementedError("write your pallas kernel here")


# speedup vs baseline: 5.0297x; 5.0297x over previous
import jax
import jax.numpy as jnp
from jax.experimental import pallas as pl
from jax.experimental.pallas import tpu as pltpu

B, P, C, O = 32, 16320, 21, 50
OP = 64
NCH = 8
BLK = P // NCH
THRESHOLD = 0.5
NEGPOS_RATIO = 3
VAR0, VAR1 = 0.1, 0.2


def _truth_rows(t):
    return (t[0:1, :], t[1:2, :], t[2:3, :], t[3:4, :], t[4:5, :],
            t[5:6, :] > 0.5)


def _match_a(t8_ref, pr_ref, bto_ref, bti_ref, bpi_ref, bpo_s, bpi_s):
    c = pl.program_id(1)
    tx1, ty1, tx2, ty2, _, valid = _truth_rows(t8_ref[0])
    area_a = (tx2 - tx1) * (ty2 - ty1)
    pr = pr_ref[...]
    pw = pr[:, 2:3]
    ph = pr[:, 3:4]
    px1 = pr[:, 0:1] - pw * 0.5
    px2 = pr[:, 0:1] + pw * 0.5
    py1 = pr[:, 1:2] - ph * 0.5
    py2 = pr[:, 1:2] + ph * 0.5
    iw = jnp.maximum(jnp.minimum(px2, tx2) - jnp.maximum(px1, tx1), 0.0)
    ih = jnp.maximum(jnp.minimum(py2, ty2) - jnp.maximum(py1, ty1), 0.0)
    inter = iw * ih
    ov = inter / (area_a + pw * ph - inter)
    ov = jnp.where(valid, ov, -1.0)
    bto_ref[0] = jnp.max(ov, axis=1, keepdims=True)
    bti_ref[0] = jnp.argmax(ov, axis=1, keepdims=True).astype(jnp.int32)

    @pl.when(c == 0)
    def _():
        bpo_s[...] = jnp.full((1, OP), -2.0, jnp.float32)
        bpi_s[...] = jnp.zeros((1, OP), jnp.int32)

    mx = jnp.max(ov, axis=0, keepdims=True)
    amx = jnp.argmax(ov, axis=0, keepdims=True).astype(jnp.int32) + c * BLK
    upd = mx > bpo_s[...]
    bpi_new = jnp.where(upd, amx, bpi_s[...])
    bpo_s[...] = jnp.where(upd, mx, bpo_s[...])
    bpi_s[...] = bpi_new
    bpi_ref[0] = bpi_new


def _match_b(t8_ref, pr_ref, loc_ref, conf_ref, bto_ref, bti_ref, bpi_ref,
             lc_ref, stats_ref):
    c = pl.program_id(1)
    tx1, ty1, tx2, ty2, lab, valid = _truth_rows(t8_ref[0])
    bpi = bpi_ref[0]
    pidx = jax.lax.broadcasted_iota(jnp.int32, (BLK, OP), 0) + c * BLK
    oid = jax.lax.broadcasted_iota(jnp.int32, (BLK, OP), 1)
    hit = (bpi == pidx) & valid
    last_j = jnp.max(jnp.where(hit, oid, -1), axis=1, keepdims=True)
    forced = last_j >= 0
    bto = jnp.where(forced, 2.0, bto_ref[0])
    bti = jnp.where(forced, last_j, bti_ref[0])
    oh = (oid == bti).astype(jnp.float32)
    gx1 = jnp.sum(oh * tx1, axis=1, keepdims=True)
    gy1 = jnp.sum(oh * ty1, axis=1, keepdims=True)
    gx2 = jnp.sum(oh * tx2, axis=1, keepdims=True)
    gy2 = jnp.sum(oh * ty2, axis=1, keepdims=True)
    glab = jnp.sum(oh * lab, axis=1, keepdims=True)
    conf_t = jnp.where(bto < THRESHOLD, 0, glab.astype(jnp.int32))
    pos = conf_t > 0

    pr = pr_ref[...]
    pw = pr[:, 2:3]
    ph = pr[:, 3:4]
    ecx = ((gx1 + gx2) * 0.5 - pr[:, 0:1]) / (VAR0 * pw)
    ecy = ((gy1 + gy2) * 0.5 - pr[:, 1:2]) / (VAR0 * ph)
    ew = jnp.log(jnp.maximum((gx2 - gx1) / pw, 1e-30)) / VAR1
    eh = jnp.log(jnp.maximum((gy2 - gy1) / ph, 1e-30)) / VAR1
    loc = loc_ref[0]

    def huber(d):
        ad = jnp.abs(d)
        return jnp.where(ad < 1.0, 0.5 * d * d, ad - 0.5)

    sl1 = (huber(loc[:, 0:1] - ecx) + huber(loc[:, 1:2] - ecy)
           + huber(loc[:, 2:3] - ew) + huber(loc[:, 3:4] - eh))
    loss_l = jnp.sum(jnp.where(pos, sl1, 0.0))
    npos = jnp.sum(pos.astype(jnp.float32))

    cf = conf_ref[0]
    m = jnp.max(cf, axis=1, keepdims=True)
    lse = jnp.log(jnp.sum(jnp.exp(cf - m), axis=1, keepdims=True)) + m
    cid = jax.lax.broadcasted_iota(jnp.int32, (BLK, C), 1)
    gat = jnp.sum(jnp.where(cid == conf_t, cf, 0.0), axis=1, keepdims=True)
    ce = lse - gat
    pos_ce = jnp.sum(jnp.where(pos, ce, 0.0))
    lc_ref[0] = jnp.where(pos, 0.0, ce)

    part = jnp.concatenate([
        loss_l.reshape(1, 1), pos_ce.reshape(1, 1), npos.reshape(1, 1),
        jnp.zeros((1, 5), jnp.float32)], axis=1)

    @pl.when(c == 0)
    def _():
        stats_ref[0] = jnp.zeros((1, 8), jnp.float32)

    stats_ref[0] = stats_ref[0] + part


def _topk_c(lc_ref, stats_ref, out_ref):
    v = lc_ref[...]
    vbits = jax.lax.bitcast_convert_type(v, jnp.int32)
    npos = stats_ref[:, 2:3]
    k = jnp.minimum(jnp.float32(NEGPOS_RATIO) * npos, jnp.float32(P - 1))
    k = k.astype(jnp.int32)

    def body(_, carry):
        lo, hi = carry
        mid = lo + jax.lax.div(hi - lo, 2)
        cnt = jnp.sum((vbits > mid).astype(jnp.int32), axis=1, keepdims=True)
        take_hi = cnt < k
        return (jnp.where(take_hi, lo, mid + 1),
                jnp.where(take_hi, mid, hi))

    lo0 = jnp.zeros((B, 1), jnp.int32)
    hi0 = jnp.full((B, 1), 0x7F800000, jnp.int32)
    lo, _ = jax.lax.fori_loop(0, 31, body, (lo0, hi0))
    t = jax.lax.bitcast_convert_type(lo, jnp.float32)
    gt = vbits > lo
    sum_gt = jnp.sum(jnp.where(gt, v, 0.0), axis=1, keepdims=True)
    cnt_gt = jnp.sum(gt.astype(jnp.float32), axis=1, keepdims=True)
    topk = sum_gt + (k.astype(jnp.float32) - cnt_gt) * t

    n = jnp.sum(npos)
    loss_l = jnp.sum(stats_ref[:, 0:1])
    loss_c = jnp.sum(stats_ref[:, 1:2] + topk)
    out_ref[...] = jnp.concatenate([
        (loss_l / n).reshape(1), (loss_c / n).reshape(1),
        jnp.zeros((6,), jnp.float32)])


@jax.jit
def kernel(loc_data, conf_data, priors, targets):
    boxes = targets[:, :, :4]
    labels = targets[:, :, 4]
    t8 = jnp.zeros((B, 8, OP), jnp.float32)
    t8 = t8.at[:, 0:4, :O].set(jnp.transpose(boxes, (0, 2, 1)))
    t8 = t8.at[:, 4, :O].set(labels)
    t8 = t8.at[:, 5, :O].set(1.0)

    bto, bti, bpi = pl.pallas_call(
        _match_a,
        grid=(B, NCH),
        in_specs=[
            pl.BlockSpec((1, 8, OP), lambda b, c: (b, 0, 0)),
            pl.BlockSpec((BLK, 4), lambda b, c: (c, 0)),
        ],
        out_specs=[
            pl.BlockSpec((1, BLK, 1), lambda b, c: (b, c, 0)),
            pl.BlockSpec((1, BLK, 1), lambda b, c: (b, c, 0)),
            pl.BlockSpec((1, 1, OP), lambda b, c: (b, 0, 0)),
        ],
        out_shape=[
            jax.ShapeDtypeStruct((B, P, 1), jnp.float32),
            jax.ShapeDtypeStruct((B, P, 1), jnp.int32),
            jax.ShapeDtypeStruct((B, 1, OP), jnp.int32),
        ],
        scratch_shapes=[
            pltpu.VMEM((1, OP), jnp.float32),
            pltpu.VMEM((1, OP), jnp.int32),
        ],
    )(t8, priors)

    lc, stats = pl.pallas_call(
        _match_b,
        grid=(B, NCH),
        in_specs=[
            pl.BlockSpec((1, 8, OP), lambda b, c: (b, 0, 0)),
            pl.BlockSpec((BLK, 4), lambda b, c: (c, 0)),
            pl.BlockSpec((1, BLK, 4), lambda b, c: (b, c, 0)),
            pl.BlockSpec((1, BLK, C), lambda b, c: (b, c, 0)),
            pl.BlockSpec((1, BLK, 1), lambda b, c: (b, c, 0)),
            pl.BlockSpec((1, BLK, 1), lambda b, c: (b, c, 0)),
            pl.BlockSpec((1, 1, OP), lambda b, c: (b, 0, 0)),
        ],
        out_specs=[
            pl.BlockSpec((1, BLK, 1), lambda b, c: (b, c, 0)),
            pl.BlockSpec((1, 1, 8), lambda b, c: (b, 0, 0)),
        ],
        out_shape=[
            jax.ShapeDtypeStruct((B, P, 1), jnp.float32),
            jax.ShapeDtypeStruct((B, 1, 8), jnp.float32),
        ],
    )(t8, priors, loc_data, conf_data, bto, bti, bpi)

    out = pl.pallas_call(
        _topk_c,
        in_specs=[
            pl.BlockSpec((B, P), lambda: (0, 0)),
            pl.BlockSpec((B, 8), lambda: (0, 0)),
        ],
        out_specs=pl.BlockSpec((8,), lambda: (0,)),
        out_shape=jax.ShapeDtypeStruct((8,), jnp.float32),
    )(lc.reshape(B, P), stats.reshape(B, 8))
    return out[0:2]
